# baseline (speedup 1.0000x reference)
import jax
import jax.numpy as jnp
from jax import lax
from jax.experimental import pallas as pl
from jax.experimental.pallas import tpu as pltpu

B_TOK = 16384 * 200
D = 64
NT = 2048
GRID = B_TOK // NT


def _tc_body(ids_ref, tab_ref, out_ref):
    ids_blk = ids_ref[...]
    oh = (ids_blk == lax.broadcasted_iota(jnp.int32, (1, 8), 1)).astype(
        jnp.float32
    )
    out_ref[...] = jnp.dot(oh, tab_ref[...], preferred_element_type=jnp.float32)


@jax.jit
def _embed_tc(ids_col, table8):
    return pl.pallas_call(
        _tc_body,
        grid=(GRID,),
        in_specs=[
            pl.BlockSpec((NT, 1), lambda i: (i, 0)),
            pl.BlockSpec((8, D), lambda i: (0, 0)),
        ],
        out_specs=pl.BlockSpec((NT, D), lambda i: (i, 0)),
        out_shape=jax.ShapeDtypeStruct((B_TOK, D), jnp.float32),
        compiler_params=pltpu.CompilerParams(
            dimension_semantics=("arbitrary",)
        ),
    )(ids_col, table8)


def kernel(ids, table):
    b, t = ids.shape
    ids_col = ids.reshape(B_TOK, 1).astype(jnp.int32)
    table8 = jnp.pad(table, ((0, 1), (0, 0)))
    out = _embed_tc(ids_col, table8)
    return out.reshape(b, t, D)
